# Initial kernel scaffold; baseline (speedup 1.0000x reference)
import jax
import jax.numpy as jnp
from jax.experimental import pallas as pl


def kernel(x, pos, batch, params):
    raise NotImplementedError("write your pallas kernel here")


# speedup vs baseline: 4.6337x; 4.6337x over previous
import jax
import jax.numpy as jnp
from jax import lax
from jax.experimental import pallas as pl

BB = 8
N = 512
M2 = 169
M2P = 256
KNB = 64
R1SQ = 0.4 * 0.4
R2SQ = 0.6 * 0.6
BIG = 3e38
F32 = jnp.float32


def _fps_kernel(posT_ref, p1T_ref, p2T_ref):
    px, py, pz = posT_ref[0], posT_ref[1], posT_ref[2]

    def fps(qx, qy, qz, m, outw):
        iota_n = lax.broadcasted_iota(jnp.int32, qx.shape, 1)
        iota_o = lax.broadcasted_iota(jnp.int32, (BB, outw), 1)
        nl = qx.shape[1]

        def sel(ai):
            ohf = (iota_n == ai).astype(F32)
            return (jnp.sum(ohf * qx, 1, keepdims=True),
                    jnp.sum(ohf * qy, 1, keepdims=True),
                    jnp.sum(ohf * qz, 1, keepdims=True))

        lx, ly, lz = sel(jnp.zeros((BB, 1), jnp.int32))
        pad = jnp.full((BB, outw), 1e9, F32)
        ox = jnp.where(iota_o == 0, lx, pad)
        oy = jnp.where(iota_o == 0, ly, pad)
        oz = jnp.where(iota_o == 0, lz, pad)
        d = (qx - lx) ** 2 + (qy - ly) ** 2 + (qz - lz) ** 2

        def body(i, st):
            d, ox, oy, oz = st
            mx = jnp.max(d, axis=1, keepdims=True)
            ai = jnp.min(jnp.where(d == mx, iota_n, nl), axis=1, keepdims=True)
            lx, ly, lz = sel(ai)
            ox = jnp.where(iota_o == i, lx, ox)
            oy = jnp.where(iota_o == i, ly, oy)
            oz = jnp.where(iota_o == i, lz, oz)
            dd = (qx - lx) ** 2 + (qy - ly) ** 2 + (qz - lz) ** 2
            return (jnp.minimum(d, dd), ox, oy, oz)

        _, ox, oy, oz = lax.fori_loop(1, m, body, (d, ox, oy, oz))
        return ox, oy, oz

    o1x, o1y, o1z = fps(px, py, pz, N, N)
    p1T_ref[0], p1T_ref[1], p1T_ref[2] = o1x, o1y, o1z
    o2x, o2y, o2z = fps(o1x, o1y, o1z, M2, M2P)
    p2T_ref[0], p2T_ref[1], p2T_ref[2] = o2x, o2y, o2z


def _sa1_kernel(posT_ref, xT_ref, p1_ref, w1_ref, b1_ref, w2_ref, b2_ref,
                w3_ref, b3_ref, out_ref):
    posT = posT_ref[0]
    px, py, pz = posT[0:1, :], posT[1:2, :], posT[2:3, :]
    xrow = xT_ref[0]
    c = p1_ref[0]
    cx, cy, cz = c[:, 0:1], c[:, 1:2], c[:, 2:3]
    ch = c.shape[0]
    d2 = (cx - px) ** 2 + (cy - py) ** 2 + (cz - pz) ** 2
    d2 = jnp.where(d2 <= R1SQ, d2, BIG)
    iota_n = lax.broadcasted_iota(jnp.int32, (ch, N), 1)

    def body(_, st):
        d2m, acc = st
        mx = jnp.min(d2m, axis=1, keepdims=True)
        ai = jnp.min(jnp.where(d2m == mx, iota_n, N), axis=1, keepdims=True)
        oh = iota_n == ai
        d2m = jnp.where(oh, BIG, d2m)
        ohf = oh.astype(F32)
        gx = jnp.sum(ohf * xrow, 1, keepdims=True)
        rx = jnp.sum(ohf * px, 1, keepdims=True) - cx
        ry = jnp.sum(ohf * py, 1, keepdims=True) - cy
        rz = jnp.sum(ohf * pz, 1, keepdims=True) - cz
        h = jnp.maximum(gx * w1_ref[0:1, :] + rx * w1_ref[1:2, :]
                        + ry * w1_ref[2:3, :] + rz * w1_ref[3:4, :]
                        + b1_ref[0:1, :], 0.0)
        h = jnp.maximum(jnp.dot(h, w2_ref[...], preferred_element_type=F32)
                        + b2_ref[0:1, :], 0.0)
        msg = jnp.maximum(jnp.dot(h, w3_ref[...], preferred_element_type=F32)
                          + b3_ref[0:1, :], 0.0)
        acc = jnp.maximum(acc, jnp.where(mx <= R1SQ, msg, -BIG))
        return (d2m, acc)

    acc0 = jnp.full((ch, 128), -BIG, F32)
    _, acc = lax.fori_loop(0, KNB, body, (d2, acc0))
    out_ref[0] = acc


def _sa2_kernel(p1T_ref, x1_ref, p2_ref, w1x_ref, w1r_ref, b1_ref, w2_ref,
                b2_ref, w3_ref, b3_ref, out_ref):
    p1T = p1T_ref[0]
    px, py, pz = p1T[0:1, :], p1T[1:2, :], p1T[2:3, :]
    y = jnp.dot(x1_ref[0], w1x_ref[...], preferred_element_type=F32)
    c = p2_ref[0]
    cx, cy, cz = c[:, 0:1], c[:, 1:2], c[:, 2:3]
    ch = c.shape[0]
    d2 = (cx - px) ** 2 + (cy - py) ** 2 + (cz - pz) ** 2
    d2 = jnp.where(d2 <= R2SQ, d2, BIG)
    iota_n = lax.broadcasted_iota(jnp.int32, (ch, N), 1)

    def body(_, st):
        d2m, acc = st
        mx = jnp.min(d2m, axis=1, keepdims=True)
        ai = jnp.min(jnp.where(d2m == mx, iota_n, N), axis=1, keepdims=True)
        oh = iota_n == ai
        d2m = jnp.where(oh, BIG, d2m)
        ohf = oh.astype(F32)
        gy = jnp.dot(ohf, y, preferred_element_type=F32)
        rx = jnp.sum(ohf * px, 1, keepdims=True) - cx
        ry = jnp.sum(ohf * py, 1, keepdims=True) - cy
        rz = jnp.sum(ohf * pz, 1, keepdims=True) - cz
        h = jnp.maximum(gy + rx * w1r_ref[0:1, :] + ry * w1r_ref[1:2, :]
                        + rz * w1r_ref[2:3, :] + b1_ref[0:1, :], 0.0)
        h = jnp.maximum(jnp.dot(h, w2_ref[...], preferred_element_type=F32)
                        + b2_ref[0:1, :], 0.0)
        msg = jnp.maximum(jnp.dot(h, w3_ref[...], preferred_element_type=F32)
                          + b3_ref[0:1, :], 0.0)
        acc = jnp.maximum(acc, jnp.where(mx <= R2SQ, msg, -BIG))
        return (d2m, acc)

    acc0 = jnp.full((ch, 256), -BIG, F32)
    _, acc = lax.fori_loop(0, KNB, body, (d2, acc0))
    row = lax.broadcasted_iota(jnp.int32, (ch, 1), 0) + pl.program_id(1) * ch
    out_ref[0] = jnp.where(row < M2, acc, 0.0)


def _knn3_interp(tx, ty, tz, sx, sy, sz, feats):
    d2 = (tx - sx) ** 2 + (ty - sy) ** 2 + (tz - sz) ** 2
    t, s = d2.shape
    iota_s = lax.broadcasted_iota(jnp.int32, (t, s), 1)
    accw = jnp.zeros((t, feats.shape[1]), F32)
    wsum = jnp.zeros((t, 1), F32)
    for _ in range(3):
        mx = jnp.min(d2, axis=1, keepdims=True)
        ai = jnp.min(jnp.where(d2 == mx, iota_s, s), axis=1, keepdims=True)
        oh = iota_s == ai
        d2 = jnp.where(oh, BIG, d2)
        w = 1.0 / jnp.maximum(mx, 1e-16)
        g = jnp.dot(oh.astype(F32), feats, preferred_element_type=F32)
        accw = accw + w * g
        wsum = wsum + w
    return accw / wsum


def _dec_kernel(x2_ref, p2_ref, p2T_ref, p1_ref, p1T_ref, x1_ref, pos_ref,
                xb_ref, w3a_ref, w3r_ref, b31_ref, w32_ref, b32_ref, w33_ref,
                b33_ref, wf3a_ref, wf3b_ref, bf31_ref, wf32_ref, bf32_ref,
                wf2a_ref, wf2b_ref, bf21_ref, wf22_ref, bf22_ref, wf1a_ref,
                wf1b_ref, bf11_ref, wf12_ref, bf12_ref, out_ref):
    x2 = x2_ref[0]
    p2 = p2_ref[0]
    h = jnp.maximum(jnp.dot(x2, w3a_ref[...], preferred_element_type=F32)
                    + p2[:, 0:1] * w3r_ref[0:1, :] + p2[:, 1:2] * w3r_ref[1:2, :]
                    + p2[:, 2:3] * w3r_ref[2:3, :] + b31_ref[0:1, :], 0.0)
    h = jnp.maximum(jnp.dot(h, w32_ref[...], preferred_element_type=F32)
                    + b32_ref[0:1, :], 0.0)
    h3 = jnp.maximum(jnp.dot(h, w33_ref[...], preferred_element_type=F32)
                     + b33_ref[0:1, :], 0.0)
    rmask = lax.broadcasted_iota(jnp.int32, (M2P, 1), 0) < M2
    x3 = jnp.max(jnp.where(rmask, h3, -BIG), axis=0, keepdims=True)
    v3 = jnp.dot(x3, wf3a_ref[...], preferred_element_type=F32)
    f3 = jnp.maximum(jnp.dot(x2, wf3b_ref[...], preferred_element_type=F32)
                     + v3 + bf31_ref[0:1, :], 0.0)
    f3 = jnp.maximum(jnp.dot(f3, wf32_ref[...], preferred_element_type=F32)
                     + bf32_ref[0:1, :], 0.0)
    p1 = p1_ref[0]
    p2T = p2T_ref[0]
    xi2 = _knn3_interp(p1[:, 0:1], p1[:, 1:2], p1[:, 2:3],
                       p2T[0:1, :], p2T[1:2, :], p2T[2:3, :], f3)
    f2 = jnp.maximum(jnp.dot(xi2, wf2a_ref[...], preferred_element_type=F32)
                     + jnp.dot(x1_ref[0], wf2b_ref[...], preferred_element_type=F32)
                     + bf21_ref[0:1, :], 0.0)
    f2 = jnp.maximum(jnp.dot(f2, wf22_ref[...], preferred_element_type=F32)
                     + bf22_ref[0:1, :], 0.0)
    pb = pos_ref[0]
    p1T = p1T_ref[0]
    xi1 = _knn3_interp(pb[:, 0:1], pb[:, 1:2], pb[:, 2:3],
                       p1T[0:1, :], p1T[1:2, :], p1T[2:3, :], f2)
    f1 = jnp.maximum(jnp.dot(xi1, wf1a_ref[...], preferred_element_type=F32)
                     + xb_ref[0] * wf1b_ref[0:1, :] + bf11_ref[0:1, :], 0.0)
    f1 = jnp.maximum(jnp.dot(f1, wf12_ref[...], preferred_element_type=F32)
                     + bf12_ref[0:1, :], 0.0)
    out_ref[0] = f1


def kernel(x, pos, batch, params):
    del batch
    pb = pos.reshape(BB, N, 3)
    xb = x.reshape(BB, N, 1)
    posT = pb.transpose(0, 2, 1)
    xT = xb.transpose(0, 2, 1)

    p1T3, p2T3 = pl.pallas_call(
        _fps_kernel,
        out_shape=[jax.ShapeDtypeStruct((3, BB, N), F32),
                   jax.ShapeDtypeStruct((3, BB, M2P), F32)],
    )(pb.transpose(2, 0, 1))
    p1 = p1T3.transpose(1, 2, 0)
    p1T = p1T3.transpose(1, 0, 2)
    p2 = p2T3.transpose(1, 2, 0)
    p2T = p2T3.transpose(1, 0, 2)

    r2 = lambda b: b.reshape(1, -1)
    (w11, b11), (w12, b12), (w13, b13) = params['sa1']
    x1 = pl.pallas_call(
        _sa1_kernel,
        grid=(BB, 4),
        in_specs=[
            pl.BlockSpec((1, 3, N), lambda b, c: (b, 0, 0)),
            pl.BlockSpec((1, 1, N), lambda b, c: (b, 0, 0)),
            pl.BlockSpec((1, 128, 3), lambda b, c: (b, c, 0)),
            pl.BlockSpec((4, 64), lambda b, c: (0, 0)),
            pl.BlockSpec((1, 64), lambda b, c: (0, 0)),
            pl.BlockSpec((64, 64), lambda b, c: (0, 0)),
            pl.BlockSpec((1, 64), lambda b, c: (0, 0)),
            pl.BlockSpec((64, 128), lambda b, c: (0, 0)),
            pl.BlockSpec((1, 128), lambda b, c: (0, 0)),
        ],
        out_specs=pl.BlockSpec((1, 128, 128), lambda b, c: (b, c, 0)),
        out_shape=jax.ShapeDtypeStruct((BB, N, 128), F32),
    )(posT, xT, p1, w11, r2(b11), w12, r2(b12), w13, r2(b13))

    (w21, b21), (w22, b22), (w23, b23) = params['sa2']
    x2 = pl.pallas_call(
        _sa2_kernel,
        grid=(BB, 2),
        in_specs=[
            pl.BlockSpec((1, 3, N), lambda b, c: (b, 0, 0)),
            pl.BlockSpec((1, N, 128), lambda b, c: (b, 0, 0)),
            pl.BlockSpec((1, 128, 3), lambda b, c: (b, c, 0)),
            pl.BlockSpec((128, 128), lambda b, c: (0, 0)),
            pl.BlockSpec((3, 128), lambda b, c: (0, 0)),
            pl.BlockSpec((1, 128), lambda b, c: (0, 0)),
            pl.BlockSpec((128, 128), lambda b, c: (0, 0)),
            pl.BlockSpec((1, 128), lambda b, c: (0, 0)),
            pl.BlockSpec((128, 256), lambda b, c: (0, 0)),
            pl.BlockSpec((1, 256), lambda b, c: (0, 0)),
        ],
        out_specs=pl.BlockSpec((1, 128, 256), lambda b, c: (b, c, 0)),
        out_shape=jax.ShapeDtypeStruct((BB, M2P, 256), F32),
    )(p1T, x1, p2, w21[:128], w21[128:131], r2(b21), w22, r2(b22), w23,
      r2(b23))

    (w31, b31), (w32, b32), (w33, b33) = params['sa3']
    (wf31, bf31), (wf32, bf32) = params['fp3']
    (wf21, bf21), (wf22, bf22) = params['fp2']
    (wf11, bf11), (wf12, bf12) = params['fp1']
    wspec = lambda s: pl.BlockSpec(s, lambda b: (0,) * len(s))
    f1 = pl.pallas_call(
        _dec_kernel,
        grid=(BB,),
        in_specs=[
            pl.BlockSpec((1, M2P, 256), lambda b: (b, 0, 0)),
            pl.BlockSpec((1, M2P, 3), lambda b: (b, 0, 0)),
            pl.BlockSpec((1, 3, M2P), lambda b: (b, 0, 0)),
            pl.BlockSpec((1, N, 3), lambda b: (b, 0, 0)),
            pl.BlockSpec((1, 3, N), lambda b: (b, 0, 0)),
            pl.BlockSpec((1, N, 128), lambda b: (b, 0, 0)),
            pl.BlockSpec((1, N, 3), lambda b: (b, 0, 0)),
            pl.BlockSpec((1, N, 1), lambda b: (b, 0, 0)),
            wspec((256, 256)), wspec((3, 256)), wspec((1, 256)),
            wspec((256, 256)), wspec((1, 256)), wspec((256, 512)),
            wspec((1, 512)),
            wspec((512, 256)), wspec((256, 256)), wspec((1, 256)),
            wspec((256, 256)), wspec((1, 256)),
            wspec((256, 128)), wspec((128, 128)), wspec((1, 128)),
            wspec((128, 128)), wspec((1, 128)),
            wspec((128, 128)), wspec((1, 128)), wspec((1, 128)),
            wspec((128, 128)), wspec((1, 128)),
        ],
        out_specs=pl.BlockSpec((1, N, 128), lambda b: (b, 0, 0)),
        out_shape=jax.ShapeDtypeStruct((BB, N, 128), F32),
    )(x2, p2, p2T, p1, p1T, x1, pb, xb,
      w31[:256], w31[256:259], r2(b31), w32, r2(b32), w33, r2(b33),
      wf31[:512], wf31[512:768], r2(bf31), wf32, r2(bf32),
      wf21[:256], wf21[256:384], r2(bf21), wf22, r2(bf22),
      wf11[:128], wf11[128:129], r2(bf11), wf12, r2(bf12))
    return f1.reshape(BB * N, 128)


# speedup vs baseline: 9.2513x; 1.9965x over previous
import jax
import jax.numpy as jnp
from jax import lax
from jax.experimental import pallas as pl

BB = 8
N = 512
M2 = 169
M2P = 176
KNB = 64
R1SQ = 0.4 * 0.4
R2SQ = 0.6 * 0.6
BIG = 3e38
F32 = jnp.float32


def _fps_kernel(posT_ref, p2T_ref):
    qx, qy, qz = posT_ref[0], posT_ref[1], posT_ref[2]
    iota_n = lax.broadcasted_iota(jnp.int32, (BB, N), 1)
    iota_o = lax.broadcasted_iota(jnp.int32, (BB, M2P), 1)

    def sel(ai):
        ohf = (iota_n == ai).astype(F32)
        return (jnp.sum(ohf * qx, 1, keepdims=True),
                jnp.sum(ohf * qy, 1, keepdims=True),
                jnp.sum(ohf * qz, 1, keepdims=True))

    lx, ly, lz = sel(jnp.zeros((BB, 1), jnp.int32))
    pad = jnp.full((BB, M2P), 1e9, F32)
    ox = jnp.where(iota_o == 0, lx, pad)
    oy = jnp.where(iota_o == 0, ly, pad)
    oz = jnp.where(iota_o == 0, lz, pad)
    d = (qx - lx) ** 2 + (qy - ly) ** 2 + (qz - lz) ** 2

    def body(i, st):
        d, ox, oy, oz = st
        mx = jnp.max(d, axis=1, keepdims=True)
        ai = jnp.min(jnp.where(d == mx, iota_n, N), axis=1, keepdims=True)
        lx, ly, lz = sel(ai)
        ox = jnp.where(iota_o == i, lx, ox)
        oy = jnp.where(iota_o == i, ly, oy)
        oz = jnp.where(iota_o == i, lz, oz)
        dd = (qx - lx) ** 2 + (qy - ly) ** 2 + (qz - lz) ** 2
        return (jnp.minimum(d, dd), ox, oy, oz)

    _, ox, oy, oz = lax.fori_loop(1, M2, body, (d, ox, oy, oz))
    p2T_ref[0], p2T_ref[1], p2T_ref[2] = ox, oy, oz


def _sa1_kernel(posT_ref, pos_ref, x_ref, w1_ref, b1_ref, w2_ref, b2_ref,
                w3_ref, b3_ref, out_ref):
    posT = posT_ref[0]
    px, py, pz = posT[0:1, :], posT[1:2, :], posT[2:3, :]
    prm = pos_ref[0]
    cx, cy, cz = prm[:, 0:1], prm[:, 1:2], prm[:, 2:3]
    xp = jnp.concatenate([x_ref[0], prm], axis=1)
    d2 = (cx - px) ** 2 + (cy - py) ** 2 + (cz - pz) ** 2
    d2 = jnp.where(d2 <= R1SQ, d2, BIG)
    iota_n = lax.broadcasted_iota(jnp.int32, (N, N), 1)
    iota_k = lax.broadcasted_iota(jnp.int32, (N, KNB), 1)

    def extract(k, st):
        d2m, gx, gpx, gpy, gpz, gm = st
        mx = jnp.min(d2m, axis=1, keepdims=True)
        ai = jnp.min(jnp.where(d2m == mx, iota_n, N), axis=1, keepdims=True)
        oh = iota_n == ai
        d2m = jnp.where(oh, BIG, d2m)
        g = jnp.dot(oh.astype(F32), xp, preferred_element_type=F32)
        colk = iota_k == k
        gx = jnp.where(colk, g[:, 0:1], gx)
        gpx = jnp.where(colk, g[:, 1:2], gpx)
        gpy = jnp.where(colk, g[:, 2:3], gpy)
        gpz = jnp.where(colk, g[:, 3:4], gpz)
        gm = jnp.where(colk, mx, gm)
        return (d2m, gx, gpx, gpy, gpz, gm)

    z = jnp.zeros((N, KNB), F32)
    _, gx, gpx, gpy, gpz, gm = lax.fori_loop(0, KNB, extract,
                                             (d2, z, z, z, z, z))

    acc = jnp.full((N, 128), -BIG, F32)
    for k in range(KNB):
        mxv = gm[:, k:k + 1]
        h = jnp.maximum(gx[:, k:k + 1] * w1_ref[0:1, :]
                        + (gpx[:, k:k + 1] - cx) * w1_ref[1:2, :]
                        + (gpy[:, k:k + 1] - cy) * w1_ref[2:3, :]
                        + (gpz[:, k:k + 1] - cz) * w1_ref[3:4, :]
                        + b1_ref[0:1, :], 0.0)
        h = jnp.maximum(jnp.dot(h, w2_ref[...], preferred_element_type=F32)
                        + b2_ref[0:1, :], 0.0)
        msg = jnp.maximum(jnp.dot(h, w3_ref[...], preferred_element_type=F32)
                          + b3_ref[0:1, :], 0.0)
        acc = jnp.maximum(acc, jnp.where(mxv <= R1SQ, msg, -BIG))
    out_ref[0] = acc


def _sa2_kernel(posT_ref, pos_ref, x1_ref, p2_ref, w1x_ref, w1r_ref, b1_ref,
                w2_ref, b2_ref, w3_ref, b3_ref, out_ref):
    posT = posT_ref[0]
    px, py, pz = posT[0:1, :], posT[1:2, :], posT[2:3, :]
    y = jnp.dot(x1_ref[0], w1x_ref[...], preferred_element_type=F32)
    yp = jnp.concatenate([y, pos_ref[0]], axis=1)
    c = p2_ref[0]
    cx, cy, cz = c[:, 0:1], c[:, 1:2], c[:, 2:3]
    d2 = (cx - px) ** 2 + (cy - py) ** 2 + (cz - pz) ** 2
    d2 = jnp.where(d2 <= R2SQ, d2, BIG)
    iota_n = lax.broadcasted_iota(jnp.int32, (M2P, N), 1)
    iota_k = lax.broadcasted_iota(jnp.int32, (M2P, KNB), 1)

    def extract(k, st):
        d2m, aik, gm = st
        mx = jnp.min(d2m, axis=1, keepdims=True)
        ai = jnp.min(jnp.where(d2m == mx, iota_n, N), axis=1, keepdims=True)
        d2m = jnp.where(iota_n == ai, BIG, d2m)
        colk = iota_k == k
        aik = jnp.where(colk, ai, aik)
        gm = jnp.where(colk, mx, gm)
        return (d2m, aik, gm)

    aik = jnp.zeros((M2P, KNB), jnp.int32)
    gm = jnp.zeros((M2P, KNB), F32)
    _, aik, gm = lax.fori_loop(0, KNB, extract, (d2, aik, gm))

    acc = jnp.full((M2P, 256), -BIG, F32)
    for k in range(KNB):
        ai = aik[:, k:k + 1]
        mxv = gm[:, k:k + 1]
        ohf = (iota_n == ai).astype(F32)
        g = jnp.dot(ohf, yp, preferred_element_type=F32)
        h = jnp.maximum(g[:, 0:128]
                        + (g[:, 128:129] - cx) * w1r_ref[0:1, :]
                        + (g[:, 129:130] - cy) * w1r_ref[1:2, :]
                        + (g[:, 130:131] - cz) * w1r_ref[2:3, :]
                        + b1_ref[0:1, :], 0.0)
        h = jnp.maximum(jnp.dot(h, w2_ref[...], preferred_element_type=F32)
                        + b2_ref[0:1, :], 0.0)
        msg = jnp.maximum(jnp.dot(h, w3_ref[...], preferred_element_type=F32)
                          + b3_ref[0:1, :], 0.0)
        acc = jnp.maximum(acc, jnp.where(mxv <= R2SQ, msg, -BIG))
    row = lax.broadcasted_iota(jnp.int32, (M2P, 1), 0)
    out_ref[0] = jnp.where(row < M2, acc, 0.0)


def _knn3_interp(tx, ty, tz, sx, sy, sz, feats):
    d2 = (tx - sx) ** 2 + (ty - sy) ** 2 + (tz - sz) ** 2
    t, s = d2.shape
    iota_s = lax.broadcasted_iota(jnp.int32, (t, s), 1)
    accw = jnp.zeros((t, feats.shape[1]), F32)
    wsum = jnp.zeros((t, 1), F32)
    for _ in range(3):
        mx = jnp.min(d2, axis=1, keepdims=True)
        ai = jnp.min(jnp.where(d2 == mx, iota_s, s), axis=1, keepdims=True)
        oh = iota_s == ai
        d2 = jnp.where(oh, BIG, d2)
        w = 1.0 / jnp.maximum(mx, 1e-16)
        g = jnp.dot(oh.astype(F32), feats, preferred_element_type=F32)
        accw = accw + w * g
        wsum = wsum + w
    return accw / wsum


def _dec_kernel(x2_ref, p2_ref, p2T_ref, p1_ref, p1T_ref, x1_ref,
                xb_ref, w3a_ref, w3r_ref, b31_ref, w32_ref, b32_ref, w33_ref,
                b33_ref, wf3a_ref, wf3b_ref, bf31_ref, wf32_ref, bf32_ref,
                wf2a_ref, wf2b_ref, bf21_ref, wf22_ref, bf22_ref, wf1a_ref,
                wf1b_ref, bf11_ref, wf12_ref, bf12_ref, out_ref):
    x2 = x2_ref[0]
    p2 = p2_ref[0]
    h = jnp.maximum(jnp.dot(x2, w3a_ref[...], preferred_element_type=F32)
                    + p2[:, 0:1] * w3r_ref[0:1, :] + p2[:, 1:2] * w3r_ref[1:2, :]
                    + p2[:, 2:3] * w3r_ref[2:3, :] + b31_ref[0:1, :], 0.0)
    h = jnp.maximum(jnp.dot(h, w32_ref[...], preferred_element_type=F32)
                    + b32_ref[0:1, :], 0.0)
    h3 = jnp.maximum(jnp.dot(h, w33_ref[...], preferred_element_type=F32)
                     + b33_ref[0:1, :], 0.0)
    rmask = lax.broadcasted_iota(jnp.int32, (M2P, 1), 0) < M2
    x3 = jnp.max(jnp.where(rmask, h3, -BIG), axis=0, keepdims=True)
    v3 = jnp.dot(x3, wf3a_ref[...], preferred_element_type=F32)
    f3 = jnp.maximum(jnp.dot(x2, wf3b_ref[...], preferred_element_type=F32)
                     + v3 + bf31_ref[0:1, :], 0.0)
    f3 = jnp.maximum(jnp.dot(f3, wf32_ref[...], preferred_element_type=F32)
                     + bf32_ref[0:1, :], 0.0)
    p1 = p1_ref[0]
    p2T = p2T_ref[0]
    xi2 = _knn3_interp(p1[:, 0:1], p1[:, 1:2], p1[:, 2:3],
                       p2T[0:1, :], p2T[1:2, :], p2T[2:3, :], f3)
    f2 = jnp.maximum(jnp.dot(xi2, wf2a_ref[...], preferred_element_type=F32)
                     + jnp.dot(x1_ref[0], wf2b_ref[...], preferred_element_type=F32)
                     + bf21_ref[0:1, :], 0.0)
    f2 = jnp.maximum(jnp.dot(f2, wf22_ref[...], preferred_element_type=F32)
                     + bf22_ref[0:1, :], 0.0)
    p1T = p1T_ref[0]
    xi1 = _knn3_interp(p1[:, 0:1], p1[:, 1:2], p1[:, 2:3],
                       p1T[0:1, :], p1T[1:2, :], p1T[2:3, :], f2)
    f1 = jnp.maximum(jnp.dot(xi1, wf1a_ref[...], preferred_element_type=F32)
                     + xb_ref[0] * wf1b_ref[0:1, :] + bf11_ref[0:1, :], 0.0)
    f1 = jnp.maximum(jnp.dot(f1, wf12_ref[...], preferred_element_type=F32)
                     + bf12_ref[0:1, :], 0.0)
    out_ref[0] = f1


def kernel(x, pos, batch, params):
    del batch
    pb = pos.reshape(BB, N, 3)
    xb = x.reshape(BB, N, 1)
    posT = pb.transpose(0, 2, 1)

    p2T3 = pl.pallas_call(
        _fps_kernel,
        out_shape=jax.ShapeDtypeStruct((3, BB, M2P), F32),
    )(pb.transpose(2, 0, 1))
    p2 = p2T3.transpose(1, 2, 0)
    p2T = p2T3.transpose(1, 0, 2)

    r2 = lambda b: b.reshape(1, -1)
    wspec = lambda s: pl.BlockSpec(s, lambda b: (0,) * len(s))
    (w11, b11), (w12, b12), (w13, b13) = params['sa1']
    x1 = pl.pallas_call(
        _sa1_kernel,
        grid=(BB,),
        in_specs=[
            pl.BlockSpec((1, 3, N), lambda b: (b, 0, 0)),
            pl.BlockSpec((1, N, 3), lambda b: (b, 0, 0)),
            pl.BlockSpec((1, N, 1), lambda b: (b, 0, 0)),
            wspec((4, 64)), wspec((1, 64)),
            wspec((64, 64)), wspec((1, 64)),
            wspec((64, 128)), wspec((1, 128)),
        ],
        out_specs=pl.BlockSpec((1, N, 128), lambda b: (b, 0, 0)),
        out_shape=jax.ShapeDtypeStruct((BB, N, 128), F32),
    )(posT, pb, xb, w11, r2(b11), w12, r2(b12), w13, r2(b13))

    (w21, b21), (w22, b22), (w23, b23) = params['sa2']
    x2 = pl.pallas_call(
        _sa2_kernel,
        grid=(BB,),
        in_specs=[
            pl.BlockSpec((1, 3, N), lambda b: (b, 0, 0)),
            pl.BlockSpec((1, N, 3), lambda b: (b, 0, 0)),
            pl.BlockSpec((1, N, 128), lambda b: (b, 0, 0)),
            pl.BlockSpec((1, M2P, 3), lambda b: (b, 0, 0)),
            wspec((128, 128)), wspec((3, 128)), wspec((1, 128)),
            wspec((128, 128)), wspec((1, 128)),
            wspec((128, 256)), wspec((1, 256)),
        ],
        out_specs=pl.BlockSpec((1, M2P, 256), lambda b: (b, 0, 0)),
        out_shape=jax.ShapeDtypeStruct((BB, M2P, 256), F32),
    )(posT, pb, x1, p2, w21[:128], w21[128:131], r2(b21), w22, r2(b22), w23,
      r2(b23))

    (w31, b31), (w32, b32), (w33, b33) = params['sa3']
    (wf31, bf31), (wf32, bf32) = params['fp3']
    (wf21, bf21), (wf22, bf22) = params['fp2']
    (wf11, bf11), (wf12, bf12) = params['fp1']
    f1 = pl.pallas_call(
        _dec_kernel,
        grid=(BB,),
        in_specs=[
            pl.BlockSpec((1, M2P, 256), lambda b: (b, 0, 0)),
            pl.BlockSpec((1, M2P, 3), lambda b: (b, 0, 0)),
            pl.BlockSpec((1, 3, M2P), lambda b: (b, 0, 0)),
            pl.BlockSpec((1, N, 3), lambda b: (b, 0, 0)),
            pl.BlockSpec((1, 3, N), lambda b: (b, 0, 0)),
            pl.BlockSpec((1, N, 128), lambda b: (b, 0, 0)),
            pl.BlockSpec((1, N, 1), lambda b: (b, 0, 0)),
            wspec((256, 256)), wspec((3, 256)), wspec((1, 256)),
            wspec((256, 256)), wspec((1, 256)), wspec((256, 512)),
            wspec((1, 512)),
            wspec((512, 256)), wspec((256, 256)), wspec((1, 256)),
            wspec((256, 256)), wspec((1, 256)),
            wspec((256, 128)), wspec((128, 128)), wspec((1, 128)),
            wspec((128, 128)), wspec((1, 128)),
            wspec((128, 128)), wspec((1, 128)), wspec((1, 128)),
            wspec((128, 128)), wspec((1, 128)),
        ],
        out_specs=pl.BlockSpec((1, N, 128), lambda b: (b, 0, 0)),
        out_shape=jax.ShapeDtypeStruct((BB, N, 128), F32),
    )(x2, p2, p2T, pb, posT, x1, xb,
      w31[:256], w31[256:259], r2(b31), w32, r2(b32), w33, r2(b33),
      wf31[:512], wf31[512:768], r2(bf31), wf32, r2(bf32),
      wf21[:256], wf21[256:384], r2(bf21), wf22, r2(bf22),
      wf11[:128], wf11[128:129], r2(bf11), wf12, r2(bf12))
    return f1.reshape(BB * N, 128)


# speedup vs baseline: 15.6512x; 1.6918x over previous
import struct

import jax
import jax.numpy as jnp
from jax import lax
from jax.experimental import pallas as pl

BB = 8
N = 512
M2 = 169
M2P = 176
KNB = 64
R1SQ = 0.4 * 0.4
R2SQ = 0.6 * 0.6
BIG = 3e38
F32 = jnp.float32

def _f32_bits(v):
    return struct.unpack('<i', struct.pack('<f', v))[0]

R1BITS = _f32_bits(R1SQ)
R2BITS = _f32_bits(R2SQ)


def _knb_threshold(d2i, rbits, rows):
    lo = jnp.zeros((rows, 1), jnp.int32)
    hi = jnp.full((rows, 1), rbits, jnp.int32)

    def bs(_, st):
        lo, hi = st
        mid = (lo + hi) // 2
        cnt = jnp.sum((d2i <= mid).astype(jnp.int32), axis=1, keepdims=True)
        ge = cnt >= KNB
        return (jnp.where(ge, lo, mid + 1), jnp.where(ge, mid, hi))

    lo, hi = lax.fori_loop(0, 31, bs, (lo, hi))
    return hi


def _fps_kernel(posT_ref, p2T_ref):
    qx, qy, qz = posT_ref[0], posT_ref[1], posT_ref[2]
    iota_n = lax.broadcasted_iota(jnp.int32, (BB, N), 1)
    iota_o = lax.broadcasted_iota(jnp.int32, (BB, M2P), 1)

    def sel(ai):
        ohf = (iota_n == ai).astype(F32)
        return (jnp.sum(ohf * qx, 1, keepdims=True),
                jnp.sum(ohf * qy, 1, keepdims=True),
                jnp.sum(ohf * qz, 1, keepdims=True))

    lx, ly, lz = sel(jnp.zeros((BB, 1), jnp.int32))
    pad = jnp.full((BB, M2P), 1e9, F32)
    ox = jnp.where(iota_o == 0, lx, pad)
    oy = jnp.where(iota_o == 0, ly, pad)
    oz = jnp.where(iota_o == 0, lz, pad)
    d = (qx - lx) ** 2 + (qy - ly) ** 2 + (qz - lz) ** 2

    def body(i, st):
        d, ox, oy, oz = st
        mx = jnp.max(d, axis=1, keepdims=True)
        ai = jnp.min(jnp.where(d == mx, iota_n, N), axis=1, keepdims=True)
        lx, ly, lz = sel(ai)
        ox = jnp.where(iota_o == i, lx, ox)
        oy = jnp.where(iota_o == i, ly, oy)
        oz = jnp.where(iota_o == i, lz, oz)
        dd = (qx - lx) ** 2 + (qy - ly) ** 2 + (qz - lz) ** 2
        return (jnp.minimum(d, dd), ox, oy, oz)

    _, ox, oy, oz = lax.fori_loop(1, M2, body, (d, ox, oy, oz))
    p2T_ref[0], p2T_ref[1], p2T_ref[2] = ox, oy, oz


def _sa1_kernel(posT_ref, pos_ref, x_ref, w1_ref, b1_ref, w2_ref, b2_ref,
                w3_ref, b3_ref, out_ref):
    posT = posT_ref[0]
    px, py, pz = posT[0:1, :], posT[1:2, :], posT[2:3, :]
    prm = pos_ref[0]
    cx, cy, cz = prm[:, 0:1], prm[:, 1:2], prm[:, 2:3]
    xp = jnp.concatenate([x_ref[0], prm], axis=1)
    d2 = (cx - px) ** 2 + (cy - py) ** 2 + (cz - pz) ** 2
    d2 = jnp.where(d2 <= R1SQ, d2, BIG)
    d2i = lax.bitcast_convert_type(d2, jnp.int32)
    t64 = _knb_threshold(d2i, R1BITS, N)
    sel = d2i <= t64
    maskf = sel.astype(F32)
    cnt = jnp.sum(maskf, axis=1, keepdims=True)
    bq = lax.broadcasted_iota(jnp.int32, (N, N), 0)
    bp = lax.broadcasted_iota(jnp.int32, (N, N), 1)
    ut = (bq < bp).astype(F32)
    rank = jnp.dot(maskf, ut, preferred_element_type=F32)
    rankm = jnp.where(sel, rank, -1.0)
    t0 = (b1_ref[0:1, :] - cx * w1_ref[1:2, :] - cy * w1_ref[2:3, :]
          - cz * w1_ref[3:4, :])
    acc = jnp.full((N, 128), -BIG, F32)
    for k in range(KNB):
        ohf = (rankm == float(k)).astype(F32)
        g = jnp.dot(ohf, xp, preferred_element_type=F32)
        h = jnp.maximum(jnp.dot(g, w1_ref[...], preferred_element_type=F32)
                        + t0, 0.0)
        h = jnp.maximum(jnp.dot(h, w2_ref[...], preferred_element_type=F32)
                        + b2_ref[0:1, :], 0.0)
        msg = jnp.maximum(jnp.dot(h, w3_ref[...], preferred_element_type=F32)
                          + b3_ref[0:1, :], 0.0)
        acc = jnp.maximum(acc, jnp.where(cnt > float(k), msg, -BIG))
    out_ref[0] = acc


def _sa2_kernel(posT_ref, pos_ref, x1_ref, p2_ref, w1x_ref, w1r_ref, b1_ref,
                w2_ref, b2_ref, w3_ref, b3_ref, out_ref):
    posT = posT_ref[0]
    px, py, pz = posT[0:1, :], posT[1:2, :], posT[2:3, :]
    yw = (jnp.dot(x1_ref[0], w1x_ref[...], preferred_element_type=F32)
          + jnp.dot(pos_ref[0], w1r_ref[...], preferred_element_type=F32))
    c = p2_ref[0]
    cx, cy, cz = c[:, 0:1], c[:, 1:2], c[:, 2:3]
    t0 = (b1_ref[0:1, :]
          - jnp.dot(c, w1r_ref[...], preferred_element_type=F32))
    d2 = (cx - px) ** 2 + (cy - py) ** 2 + (cz - pz) ** 2
    d2 = jnp.where(d2 <= R2SQ, d2, BIG)
    d2i = lax.bitcast_convert_type(d2, jnp.int32)
    t64 = _knb_threshold(d2i, R2BITS, M2P)
    sel = d2i <= t64
    maskf = sel.astype(F32)
    cnt = jnp.sum(maskf, axis=1, keepdims=True)
    bq = lax.broadcasted_iota(jnp.int32, (N, N), 0)
    bp = lax.broadcasted_iota(jnp.int32, (N, N), 1)
    ut = (bq < bp).astype(F32)
    rank = jnp.dot(maskf, ut, preferred_element_type=F32)
    rankm = jnp.where(sel, rank, -1.0)
    acc = jnp.full((M2P, 256), -BIG, F32)
    for k in range(KNB):
        ohf = (rankm == float(k)).astype(F32)
        g = jnp.dot(ohf, yw, preferred_element_type=F32)
        h = jnp.maximum(g + t0, 0.0)
        h = jnp.maximum(jnp.dot(h, w2_ref[...], preferred_element_type=F32)
                        + b2_ref[0:1, :], 0.0)
        msg = jnp.maximum(jnp.dot(h, w3_ref[...], preferred_element_type=F32)
                          + b3_ref[0:1, :], 0.0)
        acc = jnp.maximum(acc, jnp.where(cnt > float(k), msg, -BIG))
    row = lax.broadcasted_iota(jnp.int32, (M2P, 1), 0)
    out_ref[0] = jnp.where(row < M2, acc, 0.0)


def _knn3_interp(tx, ty, tz, sx, sy, sz, feats):
    d2 = (tx - sx) ** 2 + (ty - sy) ** 2 + (tz - sz) ** 2
    t, s = d2.shape
    iota_s = lax.broadcasted_iota(jnp.int32, (t, s), 1)
    accw = jnp.zeros((t, feats.shape[1]), F32)
    wsum = jnp.zeros((t, 1), F32)
    for _ in range(3):
        mx = jnp.min(d2, axis=1, keepdims=True)
        ai = jnp.min(jnp.where(d2 == mx, iota_s, s), axis=1, keepdims=True)
        oh = iota_s == ai
        d2 = jnp.where(oh, BIG, d2)
        w = 1.0 / jnp.maximum(mx, 1e-16)
        g = jnp.dot(oh.astype(F32), feats, preferred_element_type=F32)
        accw = accw + w * g
        wsum = wsum + w
    return accw / wsum


def _dec_kernel(x2_ref, p2_ref, p2T_ref, p1_ref, p1T_ref, x1_ref,
                xb_ref, w3a_ref, w3r_ref, b31_ref, w32_ref, b32_ref, w33_ref,
                b33_ref, wf3a_ref, wf3b_ref, bf31_ref, wf32_ref, bf32_ref,
                wf2a_ref, wf2b_ref, bf21_ref, wf22_ref, bf22_ref, wf1a_ref,
                wf1b_ref, bf11_ref, wf12_ref, bf12_ref, out_ref):
    x2 = x2_ref[0]
    p2 = p2_ref[0]
    h = jnp.maximum(jnp.dot(x2, w3a_ref[...], preferred_element_type=F32)
                    + p2[:, 0:1] * w3r_ref[0:1, :] + p2[:, 1:2] * w3r_ref[1:2, :]
                    + p2[:, 2:3] * w3r_ref[2:3, :] + b31_ref[0:1, :], 0.0)
    h = jnp.maximum(jnp.dot(h, w32_ref[...], preferred_element_type=F32)
                    + b32_ref[0:1, :], 0.0)
    h3 = jnp.maximum(jnp.dot(h, w33_ref[...], preferred_element_type=F32)
                     + b33_ref[0:1, :], 0.0)
    rmask = lax.broadcasted_iota(jnp.int32, (M2P, 1), 0) < M2
    x3 = jnp.max(jnp.where(rmask, h3, -BIG), axis=0, keepdims=True)
    v3 = jnp.dot(x3, wf3a_ref[...], preferred_element_type=F32)
    f3 = jnp.maximum(jnp.dot(x2, wf3b_ref[...], preferred_element_type=F32)
                     + v3 + bf31_ref[0:1, :], 0.0)
    f3 = jnp.maximum(jnp.dot(f3, wf32_ref[...], preferred_element_type=F32)
                     + bf32_ref[0:1, :], 0.0)
    p1 = p1_ref[0]
    p2T = p2T_ref[0]
    xi2 = _knn3_interp(p1[:, 0:1], p1[:, 1:2], p1[:, 2:3],
                       p2T[0:1, :], p2T[1:2, :], p2T[2:3, :], f3)
    f2 = jnp.maximum(jnp.dot(xi2, wf2a_ref[...], preferred_element_type=F32)
                     + jnp.dot(x1_ref[0], wf2b_ref[...], preferred_element_type=F32)
                     + bf21_ref[0:1, :], 0.0)
    f2 = jnp.maximum(jnp.dot(f2, wf22_ref[...], preferred_element_type=F32)
                     + bf22_ref[0:1, :], 0.0)
    p1T = p1T_ref[0]
    xi1 = _knn3_interp(p1[:, 0:1], p1[:, 1:2], p1[:, 2:3],
                       p1T[0:1, :], p1T[1:2, :], p1T[2:3, :], f2)
    f1 = jnp.maximum(jnp.dot(xi1, wf1a_ref[...], preferred_element_type=F32)
                     + xb_ref[0] * wf1b_ref[0:1, :] + bf11_ref[0:1, :], 0.0)
    f1 = jnp.maximum(jnp.dot(f1, wf12_ref[...], preferred_element_type=F32)
                     + bf12_ref[0:1, :], 0.0)
    out_ref[0] = f1


def kernel(x, pos, batch, params):
    del batch
    pb = pos.reshape(BB, N, 3)
    xb = x.reshape(BB, N, 1)
    posT = pb.transpose(0, 2, 1)

    p2T3 = pl.pallas_call(
        _fps_kernel,
        out_shape=jax.ShapeDtypeStruct((3, BB, M2P), F32),
    )(pb.transpose(2, 0, 1))
    p2 = p2T3.transpose(1, 2, 0)
    p2T = p2T3.transpose(1, 0, 2)

    r2 = lambda b: b.reshape(1, -1)
    wspec = lambda s: pl.BlockSpec(s, lambda b: (0,) * len(s))
    (w11, b11), (w12, b12), (w13, b13) = params['sa1']
    x1 = pl.pallas_call(
        _sa1_kernel,
        grid=(BB,),
        in_specs=[
            pl.BlockSpec((1, 3, N), lambda b: (b, 0, 0)),
            pl.BlockSpec((1, N, 3), lambda b: (b, 0, 0)),
            pl.BlockSpec((1, N, 1), lambda b: (b, 0, 0)),
            wspec((4, 64)), wspec((1, 64)),
            wspec((64, 64)), wspec((1, 64)),
            wspec((64, 128)), wspec((1, 128)),
        ],
        out_specs=pl.BlockSpec((1, N, 128), lambda b: (b, 0, 0)),
        out_shape=jax.ShapeDtypeStruct((BB, N, 128), F32),
    )(posT, pb, xb, w11, r2(b11), w12, r2(b12), w13, r2(b13))

    (w21, b21), (w22, b22), (w23, b23) = params['sa2']
    x2 = pl.pallas_call(
        _sa2_kernel,
        grid=(BB,),
        in_specs=[
            pl.BlockSpec((1, 3, N), lambda b: (b, 0, 0)),
            pl.BlockSpec((1, N, 3), lambda b: (b, 0, 0)),
            pl.BlockSpec((1, N, 128), lambda b: (b, 0, 0)),
            pl.BlockSpec((1, M2P, 3), lambda b: (b, 0, 0)),
            wspec((128, 128)), wspec((3, 128)), wspec((1, 128)),
            wspec((128, 128)), wspec((1, 128)),
            wspec((128, 256)), wspec((1, 256)),
        ],
        out_specs=pl.BlockSpec((1, M2P, 256), lambda b: (b, 0, 0)),
        out_shape=jax.ShapeDtypeStruct((BB, M2P, 256), F32),
    )(posT, pb, x1, p2, w21[:128], w21[128:131], r2(b21), w22, r2(b22), w23,
      r2(b23))

    (w31, b31), (w32, b32), (w33, b33) = params['sa3']
    (wf31, bf31), (wf32, bf32) = params['fp3']
    (wf21, bf21), (wf22, bf22) = params['fp2']
    (wf11, bf11), (wf12, bf12) = params['fp1']
    f1 = pl.pallas_call(
        _dec_kernel,
        grid=(BB,),
        in_specs=[
            pl.BlockSpec((1, M2P, 256), lambda b: (b, 0, 0)),
            pl.BlockSpec((1, M2P, 3), lambda b: (b, 0, 0)),
            pl.BlockSpec((1, 3, M2P), lambda b: (b, 0, 0)),
            pl.BlockSpec((1, N, 3), lambda b: (b, 0, 0)),
            pl.BlockSpec((1, 3, N), lambda b: (b, 0, 0)),
            pl.BlockSpec((1, N, 128), lambda b: (b, 0, 0)),
            pl.BlockSpec((1, N, 1), lambda b: (b, 0, 0)),
            wspec((256, 256)), wspec((3, 256)), wspec((1, 256)),
            wspec((256, 256)), wspec((1, 256)), wspec((256, 512)),
            wspec((1, 512)),
            wspec((512, 256)), wspec((256, 256)), wspec((1, 256)),
            wspec((256, 256)), wspec((1, 256)),
            wspec((256, 128)), wspec((128, 128)), wspec((1, 128)),
            wspec((128, 128)), wspec((1, 128)),
            wspec((128, 128)), wspec((1, 128)), wspec((1, 128)),
            wspec((128, 128)), wspec((1, 128)),
        ],
        out_specs=pl.BlockSpec((1, N, 128), lambda b: (b, 0, 0)),
        out_shape=jax.ShapeDtypeStruct((BB, N, 128), F32),
    )(x2, p2, p2T, pb, posT, x1, xb,
      w31[:256], w31[256:259], r2(b31), w32, r2(b32), w33, r2(b33),
      wf31[:512], wf31[512:768], r2(bf31), wf32, r2(bf32),
      wf21[:256], wf21[256:384], r2(bf21), wf22, r2(bf22),
      wf11[:128], wf11[128:129], r2(bf11), wf12, r2(bf12))
    return f1.reshape(BB * N, 128)
